# baseline (speedup 1.0000x reference)
import functools

import jax
import jax.numpy as jnp
from jax import lax
from jax.experimental import pallas as pl
from jax.experimental.pallas import tpu as pltpu
from jax.experimental.pallas import tpu_sc as plsc

N = 10000
DEG = 32
EMB = 128

NW = 32
NP = 10240
DPW = NP // NW
EPW = DPW * DEG
CG = 128
G = CG // DEG
CH = DPW // G
NB = 512
VR = EMB // 16

def _wid():
    return lax.axis_index("s") * 2 + lax.axis_index("c")


def _sc_prep_body(dbi_h, e0_h, e1_h, bs_h, nid_h, cbs_h,
                  dbi_v, e0_v, e1_v, nid_v, cbs_v, s0, s1, s2):
    wid = _wid()
    pltpu.sync_copy(dbi_h.at[wid], dbi_v)
    ebase = wid * EPW

    def chunk(j, carry):
        a = pltpu.async_copy(e0_h.at[dbi_v.at[j]], e0_v, s0)
        b = pltpu.async_copy(e1_h.at[dbi_v.at[j]], e1_v, s1)
        c = pltpu.async_copy(bs_h.at[dbi_v.at[j]], cbs_v, s2)
        a.wait()
        b.wait()
        for k in range(CG // 16):
            sl = pl.ds(k * 16, 16)
            ea = e0_v[sl]
            eb = e1_v[sl]
            ent = (ebase + j * CG + k * 16
                   + lax.broadcasted_iota(jnp.int32, (16,), 0))
            dev = jnp.right_shift(ent, 5)
            nid_v[sl] = jnp.where(ea != dev, ea, eb)
        pltpu.sync_copy(nid_v, nid_h.at[wid].at[j])
        c.wait()
        pltpu.sync_copy(cbs_v, cbs_h.at[wid].at[j])
        return carry

    lax.fori_loop(0, CH, chunk, 0)


CH_TOT = NP // G
C_A = 160
C_B = (CH_TOT // 16) - C_A
CMAX = max(C_A, C_B)
RING = 16


def _sc_gather_body(v_h, nid_h, out_h, idx_v, rows_v, ne_v, s0, s1, s2):
    cid = lax.axis_index("c")
    sid = lax.axis_index("s")
    is_a = cid == 0
    cnt = jnp.where(is_a, C_A, C_B)
    start = jnp.where(is_a, sid * C_A, 16 * C_A + sid * C_B)
    sems = (s0, s1)

    @pl.when(cnt > 0)
    def _():
        @pl.when(is_a)
        def _():
            pltpu.sync_copy(nid_h.at[pl.ds(start, C_A)],
                            idx_v.at[pl.ds(0, C_A)])

        if C_B > 0:
            @pl.when(jnp.logical_not(is_a))
            def _():
                pltpu.sync_copy(nid_h.at[pl.ds(start, C_B)],
                                idx_v.at[pl.ds(0, C_B)])

        pltpu.async_copy(v_h.at[idx_v.at[0]], rows_v.at[0], s0)
        pltpu.async_copy(v_h.at[idx_v.at[1]], rows_v.at[1], s1)

        def outer(i, carry):
            for b in range(2):
                c = i * 2 + b
                pltpu.make_async_copy(v_h.at[idx_v.at[c]], rows_v.at[b],
                                      sems[b]).wait()
                slot = jnp.bitwise_and(c, RING - 1)

                @pl.when(c >= RING)
                def _(c=c, slot=slot):
                    pltpu.make_async_copy(
                        ne_v.at[pl.ds(slot * G, G)],
                        out_h.at[pl.ds((start + c - RING) * G, G)],
                        s2).wait()

                def dev(g, inner_carry, b=b, slot=slot):
                    r0 = g * DEG
                    accs = [rows_v[b, r0, pl.ds(k * 16, 16)]
                            for k in range(VR)]
                    for d in range(1, DEG):
                        for k in range(VR):
                            accs[k] = accs[k] + rows_v[b, r0 + d,
                                                       pl.ds(k * 16, 16)]
                    row = slot * G + g
                    for k in range(VR):
                        ne_v[row, pl.ds(k * 16, 16)] = accs[k]
                    return inner_carry

                lax.fori_loop(0, G, dev, 0)
                pltpu.async_copy(ne_v.at[pl.ds(slot * G, G)],
                                 out_h.at[pl.ds((start + c) * G, G)], s2)
                nxt = c + 2

                @pl.when(nxt < cnt)
                def _(b=b, nxt=nxt):
                    pltpu.async_copy(v_h.at[idx_v.at[nxt]], rows_v.at[b],
                                     sems[b])
            return carry

        lax.fori_loop(0, cnt // 2, outer, 0)

        def drain(j, carry):
            slot = jnp.bitwise_and(j, RING - 1)
            pltpu.make_async_copy(ne_v.at[pl.ds(slot * G, G)],
                                  out_h.at[pl.ds((start + j) * G, G)],
                                  s2).wait()
            return carry

        lax.fori_loop(jnp.maximum(cnt - RING, 0), cnt, drain, 0)


@functools.cache
def _sc_kernels():
    mesh = plsc.VectorSubcoreMesh(core_axis_name="c", subcore_axis_name="s",
                                  num_cores=2, num_subcores=16)
    prep = pl.kernel(
        _sc_prep_body,
        out_type=[jax.ShapeDtypeStruct((NW, CH, CG), jnp.int32),
                  jax.ShapeDtypeStruct((NW, CH, CG), jnp.float32)],
        mesh=mesh,
        scratch_types=[pltpu.VMEM((CH, CG), jnp.int32),
                       pltpu.VMEM((CG,), jnp.int32),
                       pltpu.VMEM((CG,), jnp.int32),
                       pltpu.VMEM((CG,), jnp.int32),
                       pltpu.VMEM((CG,), jnp.float32),
                       pltpu.SemaphoreType.DMA,
                       pltpu.SemaphoreType.DMA,
                       pltpu.SemaphoreType.DMA])
    gather = pl.kernel(
        _sc_gather_body,
        out_type=jax.ShapeDtypeStruct((NP, EMB), jnp.float32),
        name="seg_gather",
        mesh=mesh,
        scratch_types=[pltpu.VMEM((CMAX, CG), jnp.int32),
                       pltpu.VMEM((2, CG, EMB), jnp.float32),
                       pltpu.VMEM((RING * G, EMB), jnp.float32),
                       pltpu.SemaphoreType.DMA,
                       pltpu.SemaphoreType.DMA,
                       pltpu.SemaphoreType.DMA])
    return prep, gather


def _tc_pre_body(convs, cbs_ref, ps_ref, w4t, b4t, w3, b3t, w1t, b1t, w2t,
                 b2t, w0, b0t, b5t, wb, ubt, f_ref, v1_ref):
    f32 = jnp.float32
    cbs = cbs_ref[...]
    sum_cbs = jnp.sum(cbs, axis=1, keepdims=True)
    be_in = jnp.zeros((cbs.shape[0], EMB), f32)
    for d in range(DEG):
        be_in = be_in + jnp.tanh(cbs[:, d:d + 1] * w4t[...] + b4t[...])
    be = jnp.tanh(
        lax.dot_general(be_in, w3[...], (((1,), (1,)), ((), ())),
                        preferred_element_type=f32) + b3t[...])
    ps = ps_ref[...]
    pe_in = jnp.zeros((cbs.shape[0], EMB), f32)
    for k in range(3):
        pe_in = pe_in + jnp.tanh(ps[:, k:k + 1] * w1t[...] + b1t[...])
    tb = jnp.tanh(sum_cbs * w2t[...] + b2t[...])
    pe = jnp.tanh(
        lax.dot_general(pe_in + 3.0 * tb, w0[...], (((1,), (1,)), ((), ())),
                        preferred_element_type=f32) + b0t[...])
    c0 = convs[0]
    c1 = convs[1]
    c2 = convs[2]
    cb = convs[3]
    f = c0 * pe + c1 * be + cb
    f_ref[...] = f
    emb = jnp.tanh(f + c2 * jnp.tanh(b5t[...]))
    g = lax.dot_general(emb, wb[...], (((1,), (1,)), ((), ())),
                        preferred_element_type=f32) + ubt[...]
    upd = g[:, EMB:2 * EMB]
    new = g[:, 2 * EMB:]
    v1_ref[...] = jnp.tanh(jax.nn.sigmoid(upd) * jnp.tanh(new))


def _tc_update_body(convs, v_ref, ne_ref, f_ref, w5, b5t, uw, ubt, vo_ref):
    f32 = jnp.float32
    ne = jnp.tanh(
        lax.dot_general(ne_ref[...], w5[...], (((1,), (1,)), ((), ())),
                        preferred_element_type=f32) + b5t[...])
    emb = jnp.tanh(f_ref[...] + convs[2] * ne)
    v = v_ref[...]
    x = jnp.concatenate([v, emb], axis=1)
    g = lax.dot_general(x, uw[...], (((1,), (1,)), ((), ())),
                        preferred_element_type=f32) + ubt[...]
    keep = g[:, :EMB]
    upd = g[:, EMB:2 * EMB]
    new = g[:, 2 * EMB:]
    vo_ref[...] = jnp.tanh(v * jax.nn.sigmoid(keep)
                           + jax.nn.sigmoid(upd) * jnp.tanh(new))


def _tc_grid_body(v_ref, gw, gbt, out_ref):
    rid = lax.broadcasted_iota(jnp.int32, (NP, 1), 0)
    vm = jnp.where(rid < N, v_ref[...], 0.0)
    parts = [jnp.sum(vm[i * 320:(i + 1) * 320], axis=0, keepdims=True)
             for i in range(32)]
    while len(parts) > 1:
        parts = [parts[i] + parts[i + 1] for i in range(0, len(parts), 2)]
    s = parts[0]
    out_ref[...] = lax.dot_general(s, gw[...], (((1,), (1,)), ((), ())),
                                   preferred_element_type=jnp.float32) + gbt[...]


def _row_spec(cols):
    return pl.BlockSpec((NB, cols), lambda i: (i, 0))


def _full_spec(shape):
    return pl.BlockSpec(shape, lambda i: tuple(0 for _ in shape))


_SMEM_SPEC = pl.BlockSpec(memory_space=pltpu.SMEM)

_GRID = NP // NB

_tc_pre = pl.pallas_call(
    _tc_pre_body,
    grid=(_GRID,),
    in_specs=[_SMEM_SPEC, _row_spec(DEG), _row_spec(3),
              _full_spec((1, EMB)), _full_spec((1, EMB)),
              _full_spec((EMB, EMB)), _full_spec((1, EMB)),
              _full_spec((1, EMB)), _full_spec((1, EMB)),
              _full_spec((1, EMB)), _full_spec((1, EMB)),
              _full_spec((EMB, EMB)), _full_spec((1, EMB)),
              _full_spec((1, EMB)), _full_spec((3 * EMB, EMB)),
              _full_spec((1, 3 * EMB))],
    out_specs=[_row_spec(EMB), _row_spec(EMB)],
    out_shape=[jax.ShapeDtypeStruct((NP, EMB), jnp.float32),
               jax.ShapeDtypeStruct((NP, EMB), jnp.float32)],
)

_tc_update = pl.pallas_call(
    _tc_update_body,
    grid=(_GRID,),
    in_specs=[_SMEM_SPEC, _row_spec(EMB), _row_spec(EMB), _row_spec(EMB),
              _full_spec((EMB, EMB)), _full_spec((1, EMB)),
              _full_spec((3 * EMB, 2 * EMB)), _full_spec((1, 3 * EMB))],
    out_specs=_row_spec(EMB),
    out_shape=jax.ShapeDtypeStruct((NP, EMB), jnp.float32),
)

_tc_grid = pl.pallas_call(
    _tc_grid_body,
    out_shape=jax.ShapeDtypeStruct((1, EMB), jnp.float32),
)


def kernel(protector_state, breaker_state, device_breaker_ids, breakers,
           W0, b0, W1, b1, W2, b2, W3, b3, W4, b4, W5, b5,
           conv_w, conv_b, update_W, update_b, grid_W, grid_b):
    f32 = jnp.float32
    dbi = device_breaker_ids.astype(jnp.int32)
    dbi_pad = jnp.pad(dbi, ((0, NP - N), (0, 0)))
    dbi_r = dbi_pad.reshape(NW, CH, CG)
    e0 = breakers[:, 0].astype(jnp.int32)
    e1 = breakers[:, 1].astype(jnp.int32)
    ps_pad = jnp.pad(protector_state.astype(f32), ((0, NP - N), (0, 0)))

    _sc_prep, _sc_gather = _sc_kernels()
    nid_r, cbs_flat = _sc_prep(dbi_r, e0, e1, breaker_state.astype(f32))
    nid_r = nid_r.reshape(CH_TOT, CG)
    cbs = cbs_flat.reshape(NP, DEG)

    w1t = W1.T.astype(f32)
    w2t = W2.T.astype(f32)
    w4t = W4.T.astype(f32)
    b0t = b0.reshape(1, EMB)
    b1t = b1.reshape(1, EMB)
    b2t = b2.reshape(1, EMB)
    b3t = b3.reshape(1, EMB)
    b4t = b4.reshape(1, EMB)
    b5t = b5.reshape(1, EMB)
    ubt = update_b.reshape(1, 3 * EMB)
    gbt = grid_b.reshape(1, EMB)
    wb = update_W[:, EMB:]
    convs = jnp.concatenate([conv_w.reshape(3), conv_b.reshape(1)]).astype(f32)

    F, V = _tc_pre(convs, cbs, ps_pad, w4t, b4t, W3, b3t, w1t, b1t, w2t,
                   b2t, W0, b0t, b5t, wb, ubt)
    for _ in range(2):
        ne_raw = _sc_gather(V, nid_r)
        V = _tc_update(convs, V, ne_raw, F, W5, b5t, update_W, ubt)
    grid_emb = _tc_grid(V, grid_W, gbt)
    return V[:N], grid_emb.reshape(EMB)


# speedup vs baseline: 1.0007x; 1.0007x over previous
import functools

import jax
import jax.numpy as jnp
from jax import lax
from jax.experimental import pallas as pl
from jax.experimental.pallas import tpu as pltpu
from jax.experimental.pallas import tpu_sc as plsc

N = 10000
DEG = 32
EMB = 128

NW = 32
NP = 10240
DPW = NP // NW
EPW = DPW * DEG
CG = 128
G = CG // DEG
CH = DPW // G
NB = 512
VR = EMB // 16

def _wid():
    return lax.axis_index("s") * 2 + lax.axis_index("c")


def _sc_prep_body(dbi_h, e0_h, e1_h, bs_h, nid_h, cbs_h,
                  dbi_v, e0_v, e1_v, nid_v, cbs_v, s0, s1, s2):
    wid = _wid()
    pltpu.sync_copy(dbi_h.at[wid], dbi_v)
    ebase = wid * EPW

    def chunk(j, carry):
        a = pltpu.async_copy(e0_h.at[dbi_v.at[j]], e0_v, s0)
        b = pltpu.async_copy(e1_h.at[dbi_v.at[j]], e1_v, s1)
        c = pltpu.async_copy(bs_h.at[dbi_v.at[j]], cbs_v, s2)
        a.wait()
        b.wait()
        for k in range(CG // 16):
            sl = pl.ds(k * 16, 16)
            ea = e0_v[sl]
            eb = e1_v[sl]
            ent = (ebase + j * CG + k * 16
                   + lax.broadcasted_iota(jnp.int32, (16,), 0))
            dev = jnp.right_shift(ent, 5)
            nid_v[sl] = jnp.where(ea != dev, ea, eb)
        pltpu.sync_copy(nid_v, nid_h.at[wid].at[j])
        c.wait()
        pltpu.sync_copy(cbs_v, cbs_h.at[wid].at[j])
        return carry

    lax.fori_loop(0, CH, chunk, 0)


CH_TOT = NP // G
C_A = 160
C_B = (CH_TOT // 16) - C_A
CMAX = max(C_A, C_B)
FLUSH = 40


def _sc_gather_body(v_h, nid_h, out_h, idx_v, rows_v, ne_v, s0, s1):
    cid = lax.axis_index("c")
    sid = lax.axis_index("s")
    is_a = cid == 0
    cnt = jnp.where(is_a, C_A, C_B)
    start = jnp.where(is_a, sid * C_A, 16 * C_A + sid * C_B)
    sems = (s0, s1)

    @pl.when(cnt > 0)
    def _():
        @pl.when(is_a)
        def _():
            pltpu.sync_copy(nid_h.at[pl.ds(start, C_A)],
                            idx_v.at[pl.ds(0, C_A)])

        if C_B > 0:
            @pl.when(jnp.logical_not(is_a))
            def _():
                pltpu.sync_copy(nid_h.at[pl.ds(start, C_B)],
                                idx_v.at[pl.ds(0, C_B)])

        pltpu.async_copy(v_h.at[idx_v.at[0]], rows_v.at[0], s0)
        pltpu.async_copy(v_h.at[idx_v.at[1]], rows_v.at[1], s1)

        def outer(i, carry):
            for b in range(2):
                c = i * 2 + b
                pltpu.make_async_copy(v_h.at[idx_v.at[c]], rows_v.at[b],
                                      sems[b]).wait()
                fc = lax.rem(c, FLUSH)

                def dev(g, inner_carry, b=b, fc=fc):
                    r0 = g * DEG
                    accs = [rows_v[b, r0, pl.ds(k * 16, 16)]
                            for k in range(VR)]
                    for d in range(1, DEG):
                        for k in range(VR):
                            accs[k] = accs[k] + rows_v[b, r0 + d,
                                                       pl.ds(k * 16, 16)]
                    row = fc * G + g
                    for k in range(VR):
                        ne_v[row, pl.ds(k * 16, 16)] = accs[k]
                    return inner_carry

                lax.fori_loop(0, G, dev, 0)
                nxt = c + 2

                @pl.when(nxt < cnt)
                def _(b=b, nxt=nxt):
                    pltpu.async_copy(v_h.at[idx_v.at[nxt]], rows_v.at[b],
                                     sems[b])

                @pl.when(fc == FLUSH - 1)
                def _(c=c):
                    off = pl.multiple_of(
                        (start + c - (FLUSH - 1)) * G, 8)
                    pltpu.sync_copy(ne_v, out_h.at[pl.ds(off, FLUSH * G)])
            return carry

        lax.fori_loop(0, cnt // 2, outer, 0)


@functools.cache
def _sc_kernels():
    mesh = plsc.VectorSubcoreMesh(core_axis_name="c", subcore_axis_name="s",
                                  num_cores=2, num_subcores=16)
    prep = pl.kernel(
        _sc_prep_body,
        out_type=[jax.ShapeDtypeStruct((NW, CH, CG), jnp.int32),
                  jax.ShapeDtypeStruct((NW, CH, CG), jnp.float32)],
        mesh=mesh,
        scratch_types=[pltpu.VMEM((CH, CG), jnp.int32),
                       pltpu.VMEM((CG,), jnp.int32),
                       pltpu.VMEM((CG,), jnp.int32),
                       pltpu.VMEM((CG,), jnp.int32),
                       pltpu.VMEM((CG,), jnp.float32),
                       pltpu.SemaphoreType.DMA,
                       pltpu.SemaphoreType.DMA,
                       pltpu.SemaphoreType.DMA])
    gather = pl.kernel(
        _sc_gather_body,
        out_type=jax.ShapeDtypeStruct((NP, EMB), jnp.float32),
        name="seg_gather",
        mesh=mesh,
        scratch_types=[pltpu.VMEM((CMAX, CG), jnp.int32),
                       pltpu.VMEM((2, CG, EMB), jnp.float32),
                       pltpu.VMEM((FLUSH * G, EMB), jnp.float32),
                       pltpu.SemaphoreType.DMA,
                       pltpu.SemaphoreType.DMA])
    return prep, gather


def _tc_pre_body(convs, cbs_ref, ps_ref, w4t, b4t, w3, b3t, w1t, b1t, w2t,
                 b2t, w0, b0t, b5t, wb, ubt, f_ref, v1_ref):
    f32 = jnp.float32
    cbs = cbs_ref[...]
    sum_cbs = jnp.sum(cbs, axis=1, keepdims=True)
    be_in = jnp.zeros((cbs.shape[0], EMB), f32)
    for d in range(DEG):
        be_in = be_in + jnp.tanh(cbs[:, d:d + 1] * w4t[...] + b4t[...])
    be = jnp.tanh(
        lax.dot_general(be_in, w3[...], (((1,), (1,)), ((), ())),
                        preferred_element_type=f32) + b3t[...])
    ps = ps_ref[...]
    pe_in = jnp.zeros((cbs.shape[0], EMB), f32)
    for k in range(3):
        pe_in = pe_in + jnp.tanh(ps[:, k:k + 1] * w1t[...] + b1t[...])
    tb = jnp.tanh(sum_cbs * w2t[...] + b2t[...])
    pe = jnp.tanh(
        lax.dot_general(pe_in + 3.0 * tb, w0[...], (((1,), (1,)), ((), ())),
                        preferred_element_type=f32) + b0t[...])
    c0 = convs[0]
    c1 = convs[1]
    c2 = convs[2]
    cb = convs[3]
    f = c0 * pe + c1 * be + cb
    f_ref[...] = f
    emb = jnp.tanh(f + c2 * jnp.tanh(b5t[...]))
    g = lax.dot_general(emb, wb[...], (((1,), (1,)), ((), ())),
                        preferred_element_type=f32) + ubt[...]
    upd = g[:, EMB:2 * EMB]
    new = g[:, 2 * EMB:]
    v1_ref[...] = jnp.tanh(jax.nn.sigmoid(upd) * jnp.tanh(new))


def _tc_update_body(convs, v_ref, ne_ref, f_ref, w5, b5t, uw, ubt, vo_ref):
    f32 = jnp.float32
    ne = jnp.tanh(
        lax.dot_general(ne_ref[...], w5[...], (((1,), (1,)), ((), ())),
                        preferred_element_type=f32) + b5t[...])
    emb = jnp.tanh(f_ref[...] + convs[2] * ne)
    v = v_ref[...]
    x = jnp.concatenate([v, emb], axis=1)
    g = lax.dot_general(x, uw[...], (((1,), (1,)), ((), ())),
                        preferred_element_type=f32) + ubt[...]
    keep = g[:, :EMB]
    upd = g[:, EMB:2 * EMB]
    new = g[:, 2 * EMB:]
    vo_ref[...] = jnp.tanh(v * jax.nn.sigmoid(keep)
                           + jax.nn.sigmoid(upd) * jnp.tanh(new))


def _tc_grid_body(v_ref, gw, gbt, out_ref):
    rid = lax.broadcasted_iota(jnp.int32, (NP, 1), 0)
    vm = jnp.where(rid < N, v_ref[...], 0.0)
    parts = [jnp.sum(vm[i * 320:(i + 1) * 320], axis=0, keepdims=True)
             for i in range(32)]
    while len(parts) > 1:
        parts = [parts[i] + parts[i + 1] for i in range(0, len(parts), 2)]
    s = parts[0]
    out_ref[...] = lax.dot_general(s, gw[...], (((1,), (1,)), ((), ())),
                                   preferred_element_type=jnp.float32) + gbt[...]


def _row_spec(cols):
    return pl.BlockSpec((NB, cols), lambda i: (i, 0))


def _full_spec(shape):
    return pl.BlockSpec(shape, lambda i: tuple(0 for _ in shape))


_SMEM_SPEC = pl.BlockSpec(memory_space=pltpu.SMEM)

_GRID = NP // NB

_tc_pre = pl.pallas_call(
    _tc_pre_body,
    grid=(_GRID,),
    in_specs=[_SMEM_SPEC, _row_spec(DEG), _row_spec(3),
              _full_spec((1, EMB)), _full_spec((1, EMB)),
              _full_spec((EMB, EMB)), _full_spec((1, EMB)),
              _full_spec((1, EMB)), _full_spec((1, EMB)),
              _full_spec((1, EMB)), _full_spec((1, EMB)),
              _full_spec((EMB, EMB)), _full_spec((1, EMB)),
              _full_spec((1, EMB)), _full_spec((3 * EMB, EMB)),
              _full_spec((1, 3 * EMB))],
    out_specs=[_row_spec(EMB), _row_spec(EMB)],
    out_shape=[jax.ShapeDtypeStruct((NP, EMB), jnp.float32),
               jax.ShapeDtypeStruct((NP, EMB), jnp.float32)],
)

_tc_update = pl.pallas_call(
    _tc_update_body,
    grid=(_GRID,),
    in_specs=[_SMEM_SPEC, _row_spec(EMB), _row_spec(EMB), _row_spec(EMB),
              _full_spec((EMB, EMB)), _full_spec((1, EMB)),
              _full_spec((3 * EMB, 2 * EMB)), _full_spec((1, 3 * EMB))],
    out_specs=_row_spec(EMB),
    out_shape=jax.ShapeDtypeStruct((NP, EMB), jnp.float32),
)

_tc_grid = pl.pallas_call(
    _tc_grid_body,
    out_shape=jax.ShapeDtypeStruct((1, EMB), jnp.float32),
)


def kernel(protector_state, breaker_state, device_breaker_ids, breakers,
           W0, b0, W1, b1, W2, b2, W3, b3, W4, b4, W5, b5,
           conv_w, conv_b, update_W, update_b, grid_W, grid_b):
    f32 = jnp.float32
    dbi = device_breaker_ids.astype(jnp.int32)
    dbi_pad = jnp.pad(dbi, ((0, NP - N), (0, 0)))
    dbi_r = dbi_pad.reshape(NW, CH, CG)
    e0 = breakers[:, 0].astype(jnp.int32)
    e1 = breakers[:, 1].astype(jnp.int32)
    ps_pad = jnp.pad(protector_state.astype(f32), ((0, NP - N), (0, 0)))

    _sc_prep, _sc_gather = _sc_kernels()
    nid_r, cbs_flat = _sc_prep(dbi_r, e0, e1, breaker_state.astype(f32))
    nid_r = nid_r.reshape(CH_TOT, CG)
    cbs = cbs_flat.reshape(NP, DEG)

    w1t = W1.T.astype(f32)
    w2t = W2.T.astype(f32)
    w4t = W4.T.astype(f32)
    b0t = b0.reshape(1, EMB)
    b1t = b1.reshape(1, EMB)
    b2t = b2.reshape(1, EMB)
    b3t = b3.reshape(1, EMB)
    b4t = b4.reshape(1, EMB)
    b5t = b5.reshape(1, EMB)
    ubt = update_b.reshape(1, 3 * EMB)
    gbt = grid_b.reshape(1, EMB)
    wb = update_W[:, EMB:]
    convs = jnp.concatenate([conv_w.reshape(3), conv_b.reshape(1)]).astype(f32)

    F, V = _tc_pre(convs, cbs, ps_pad, w4t, b4t, W3, b3t, w1t, b1t, w2t,
                   b2t, W0, b0t, b5t, wb, ubt)
    for _ in range(2):
        ne_raw = _sc_gather(V, nid_r)
        V = _tc_update(convs, V, ne_raw, F, W5, b5t, update_W, ubt)
    grid_emb = _tc_grid(V, grid_W, gbt)
    return V[:N], grid_emb.reshape(EMB)


# speedup vs baseline: 2.9483x; 2.9463x over previous
import functools

import jax
import jax.numpy as jnp
from jax import lax
from jax.experimental import pallas as pl
from jax.experimental.pallas import tpu as pltpu
from jax.experimental.pallas import tpu_sc as plsc

N = 10000
DEG = 32
EMB = 128
E_TOT = 160000

NW = 32
NP = 10240
DPW = NP // NW
EPW = DPW * DEG
CG = 128
G = CG // DEG
CH = DPW // G
NB = 512
VR = EMB // 16

def _wid():
    return lax.axis_index("s") * 2 + lax.axis_index("c")


def _sc_prep_body(dbi_h, e0_h, e1_h, bs_h, nid_h, cbs_h,
                  dbi_v, e0_v, e1_v, nid_v, cbs_v, s0, s1, s2):
    wid = _wid()
    pltpu.sync_copy(dbi_h.at[wid], dbi_v)
    ebase = wid * EPW

    def chunk(j, carry):
        a = pltpu.async_copy(e0_h.at[dbi_v.at[j]], e0_v, s0)
        b = pltpu.async_copy(e1_h.at[dbi_v.at[j]], e1_v, s1)
        c = pltpu.async_copy(bs_h.at[dbi_v.at[j]], cbs_v, s2)
        a.wait()
        b.wait()
        for k in range(CG // 16):
            sl = pl.ds(k * 16, 16)
            ea = e0_v[sl]
            eb = e1_v[sl]
            ent = (ebase + j * CG + k * 16
                   + lax.broadcasted_iota(jnp.int32, (16,), 0))
            dev = jnp.right_shift(ent, 5)
            nid_v[sl] = jnp.where(ea != dev, ea, eb)
        pltpu.sync_copy(nid_v, nid_h.at[wid].at[j])
        c.wait()
        pltpu.sync_copy(cbs_v, cbs_h.at[wid].at[j])
        return carry

    lax.fori_loop(0, CH, chunk, 0)


CH_TOT = NP // G
C_A = 80
C_B = (CH_TOT // 16) - C_A
CMAX = max(C_A, C_B)
FLUSH = 40


def _sc_gather_body(v_h, nid_h, out_h, idx_v, rows_v, ne_v, s0, s1):
    cid = lax.axis_index("c")
    sid = lax.axis_index("s")
    is_a = cid == 0
    cnt = jnp.where(is_a, C_A, C_B)
    start = jnp.where(is_a, sid * C_A, 16 * C_A + sid * C_B)
    sems = (s0, s1)

    @pl.when(cnt > 0)
    def _():
        @pl.when(is_a)
        def _():
            pltpu.sync_copy(nid_h.at[pl.ds(start, C_A)],
                            idx_v.at[pl.ds(0, C_A)])

        if C_B > 0:
            @pl.when(jnp.logical_not(is_a))
            def _():
                pltpu.sync_copy(nid_h.at[pl.ds(start, C_B)],
                                idx_v.at[pl.ds(0, C_B)])

        pltpu.async_copy(v_h.at[idx_v.at[0]], rows_v.at[0], s0)
        pltpu.async_copy(v_h.at[idx_v.at[1]], rows_v.at[1], s1)

        def outer(i, carry):
            for b in range(2):
                c = i * 2 + b
                pltpu.make_async_copy(v_h.at[idx_v.at[c]], rows_v.at[b],
                                      sems[b]).wait()
                fc = lax.rem(c, FLUSH)

                def dev(g, inner_carry, b=b, fc=fc):
                    r0 = g * DEG
                    accs = [rows_v[b, r0, pl.ds(k * 16, 16)]
                            for k in range(VR)]
                    for d in range(1, DEG):
                        for k in range(VR):
                            accs[k] = accs[k] + rows_v[b, r0 + d,
                                                       pl.ds(k * 16, 16)]
                    row = fc * G + g
                    for k in range(VR):
                        ne_v[row, pl.ds(k * 16, 16)] = accs[k]
                    return inner_carry

                lax.fori_loop(0, G, dev, 0)
                nxt = c + 2

                @pl.when(nxt < cnt)
                def _(b=b, nxt=nxt):
                    pltpu.async_copy(v_h.at[idx_v.at[nxt]], rows_v.at[b],
                                     sems[b])

                @pl.when(fc == FLUSH - 1)
                def _(c=c):
                    off = pl.multiple_of(
                        (start + c - (FLUSH - 1)) * G, 8)
                    pltpu.sync_copy(ne_v, out_h.at[pl.ds(off, FLUSH * G)])
            return carry

        lax.fori_loop(0, cnt // 2, outer, 0)


@functools.cache
def _sc_kernels():
    mesh = plsc.VectorSubcoreMesh(core_axis_name="c", subcore_axis_name="s",
                                  num_cores=2, num_subcores=16)
    prep = pl.kernel(
        _sc_prep_body,
        out_type=[jax.ShapeDtypeStruct((NW, CH, CG), jnp.int32),
                  jax.ShapeDtypeStruct((NW, CH, CG), jnp.float32)],
        mesh=mesh,
        scratch_types=[pltpu.VMEM((CH, CG), jnp.int32),
                       pltpu.VMEM((CG,), jnp.int32),
                       pltpu.VMEM((CG,), jnp.int32),
                       pltpu.VMEM((CG,), jnp.int32),
                       pltpu.VMEM((CG,), jnp.float32),
                       pltpu.SemaphoreType.DMA,
                       pltpu.SemaphoreType.DMA,
                       pltpu.SemaphoreType.DMA])
    gather = pl.kernel(
        _sc_gather_body,
        out_type=jax.ShapeDtypeStruct((NP, EMB), jnp.float32),
        name="seg_gather",
        mesh=mesh,
        scratch_types=[pltpu.VMEM((CMAX, CG), jnp.int32),
                       pltpu.VMEM((2, CG, EMB), jnp.float32),
                       pltpu.VMEM((FLUSH * G, EMB), jnp.float32),
                       pltpu.SemaphoreType.DMA,
                       pltpu.SemaphoreType.DMA])
    return prep, gather


def _tc_pre_body(convs, cbs_ref, ps_ref, w4t, b4t, w3, b3t, w1t, b1t, w2t,
                 b2t, w0, b0t, b5t, wb, ubt, f_ref, v1_ref):
    f32 = jnp.float32
    cbs = cbs_ref[...]
    sum_cbs = jnp.sum(cbs, axis=1, keepdims=True)
    be_in = jnp.zeros((cbs.shape[0], EMB), f32)
    for d in range(DEG):
        be_in = be_in + jnp.tanh(cbs[:, d:d + 1] * w4t[...] + b4t[...])
    be = jnp.tanh(
        lax.dot_general(be_in, w3[...], (((1,), (1,)), ((), ())),
                        preferred_element_type=f32) + b3t[...])
    ps = ps_ref[...]
    pe_in = jnp.zeros((cbs.shape[0], EMB), f32)
    for k in range(3):
        pe_in = pe_in + jnp.tanh(ps[:, k:k + 1] * w1t[...] + b1t[...])
    tb = jnp.tanh(sum_cbs * w2t[...] + b2t[...])
    pe = jnp.tanh(
        lax.dot_general(pe_in + 3.0 * tb, w0[...], (((1,), (1,)), ((), ())),
                        preferred_element_type=f32) + b0t[...])
    c0 = convs[0]
    c1 = convs[1]
    c2 = convs[2]
    cb = convs[3]
    f = c0 * pe + c1 * be + cb
    f_ref[...] = f
    emb = jnp.tanh(f + c2 * jnp.tanh(b5t[...]))
    g = lax.dot_general(emb, wb[...], (((1,), (1,)), ((), ())),
                        preferred_element_type=f32) + ubt[...]
    upd = g[:, EMB:2 * EMB]
    new = g[:, 2 * EMB:]
    v1_ref[...] = jnp.tanh(jax.nn.sigmoid(upd) * jnp.tanh(new))


def _tc_update_body(convs, v_ref, ne_ref, f_ref, w5, b5t, uw, ubt, vo_ref):
    f32 = jnp.float32
    ne = jnp.tanh(
        lax.dot_general(ne_ref[...], w5[...], (((1,), (1,)), ((), ())),
                        preferred_element_type=f32) + b5t[...])
    emb = jnp.tanh(f_ref[...] + convs[2] * ne)
    v = v_ref[...]
    x = jnp.concatenate([v, emb], axis=1)
    g = lax.dot_general(x, uw[...], (((1,), (1,)), ((), ())),
                        preferred_element_type=f32) + ubt[...]
    keep = g[:, :EMB]
    upd = g[:, EMB:2 * EMB]
    new = g[:, 2 * EMB:]
    vo_ref[...] = jnp.tanh(v * jax.nn.sigmoid(keep)
                           + jax.nn.sigmoid(upd) * jnp.tanh(new))


def _tc_grid_body(v_ref, gw, gbt, out_ref):
    rid = lax.broadcasted_iota(jnp.int32, (NP, 1), 0)
    vm = jnp.where(rid < N, v_ref[...], 0.0)
    parts = [jnp.sum(vm[i * 320:(i + 1) * 320], axis=0, keepdims=True)
             for i in range(32)]
    while len(parts) > 1:
        parts = [parts[i] + parts[i + 1] for i in range(0, len(parts), 2)]
    s = parts[0]
    out_ref[...] = lax.dot_general(s, gw[...], (((1,), (1,)), ((), ())),
                                   preferred_element_type=jnp.float32) + gbt[...]


def _row_spec(cols):
    return pl.BlockSpec((NB, cols), lambda i: (i, 0))


def _full_spec(shape):
    return pl.BlockSpec(shape, lambda i: tuple(0 for _ in shape))


_SMEM_SPEC = pl.BlockSpec(memory_space=pltpu.SMEM)

_GRID = NP // NB

_tc_pre = pl.pallas_call(
    _tc_pre_body,
    grid=(_GRID,),
    in_specs=[_SMEM_SPEC, _row_spec(DEG), _row_spec(3),
              _full_spec((1, EMB)), _full_spec((1, EMB)),
              _full_spec((EMB, EMB)), _full_spec((1, EMB)),
              _full_spec((1, EMB)), _full_spec((1, EMB)),
              _full_spec((1, EMB)), _full_spec((1, EMB)),
              _full_spec((EMB, EMB)), _full_spec((1, EMB)),
              _full_spec((1, EMB)), _full_spec((3 * EMB, EMB)),
              _full_spec((1, 3 * EMB))],
    out_specs=[_row_spec(EMB), _row_spec(EMB)],
    out_shape=[jax.ShapeDtypeStruct((NP, EMB), jnp.float32),
               jax.ShapeDtypeStruct((NP, EMB), jnp.float32)],
)

_tc_update = pl.pallas_call(
    _tc_update_body,
    grid=(_GRID,),
    in_specs=[_SMEM_SPEC, _row_spec(EMB), _row_spec(EMB), _row_spec(EMB),
              _full_spec((EMB, EMB)), _full_spec((1, EMB)),
              _full_spec((3 * EMB, 2 * EMB)), _full_spec((1, 3 * EMB))],
    out_specs=_row_spec(EMB),
    out_shape=jax.ShapeDtypeStruct((NP, EMB), jnp.float32),
)

_tc_grid = pl.pallas_call(
    _tc_grid_body,
    out_shape=jax.ShapeDtypeStruct((1, EMB), jnp.float32),
)


def kernel(protector_state, breaker_state, device_breaker_ids, breakers,
           W0, b0, W1, b1, W2, b2, W3, b3, W4, b4, W5, b5,
           conv_w, conv_b, update_W, update_b, grid_W, grid_b):
    f32 = jnp.float32
    dbi = device_breaker_ids.astype(jnp.int32)
    pad_ids = (jnp.arange((NP - N) * DEG, dtype=jnp.int32) * 97
               % jnp.int32(E_TOT)).reshape(NP - N, DEG)
    dbi_pad = jnp.concatenate([dbi, pad_ids], axis=0)
    dbi_r = dbi_pad.reshape(NW, CH, CG)
    e0 = breakers[:, 0].astype(jnp.int32)
    e1 = breakers[:, 1].astype(jnp.int32)
    ps_pad = jnp.pad(protector_state.astype(f32), ((0, NP - N), (0, 0)))

    _sc_prep, _sc_gather = _sc_kernels()
    nid_r, cbs_flat = _sc_prep(dbi_r, e0, e1, breaker_state.astype(f32))
    nid_r = nid_r.reshape(CH_TOT, CG)
    cbs = cbs_flat.reshape(NP, DEG)

    w1t = W1.T.astype(f32)
    w2t = W2.T.astype(f32)
    w4t = W4.T.astype(f32)
    b0t = b0.reshape(1, EMB)
    b1t = b1.reshape(1, EMB)
    b2t = b2.reshape(1, EMB)
    b3t = b3.reshape(1, EMB)
    b4t = b4.reshape(1, EMB)
    b5t = b5.reshape(1, EMB)
    ubt = update_b.reshape(1, 3 * EMB)
    gbt = grid_b.reshape(1, EMB)
    wb = update_W[:, EMB:]
    convs = jnp.concatenate([conv_w.reshape(3), conv_b.reshape(1)]).astype(f32)

    F, V = _tc_pre(convs, cbs, ps_pad, w4t, b4t, W3, b3t, w1t, b1t, w2t,
                   b2t, W0, b0t, b5t, wb, ubt)
    for _ in range(2):
        ne_raw = _sc_gather(V, nid_r)
        V = _tc_update(convs, V, ne_raw, F, W5, b5t, update_W, ubt)
    grid_emb = _tc_grid(V, grid_W, gbt)
    return V[:N], grid_emb.reshape(EMB)


# speedup vs baseline: 3.1581x; 1.0711x over previous
import functools

import jax
import jax.numpy as jnp
from jax import lax
from jax.experimental import pallas as pl
from jax.experimental.pallas import tpu as pltpu
from jax.experimental.pallas import tpu_sc as plsc

N = 10000
DEG = 32
EMB = 128
E_TOT = 160000

NW = 32
NP = 10240
DPW = NP // NW
EPW = DPW * DEG
CG = 128
G = CG // DEG
CH = DPW // G
NB = 512
VR = EMB // 16

def _wid():
    return lax.axis_index("s") * 2 + lax.axis_index("c")


def _sc_prep_body(dbi_h, e0_h, e1_h, bs_h, nid_h, cbs_h,
                  dbi_v, e0_v, e1_v, nid_v, cbs_v, s0, s1, s2):
    wid = _wid()
    pltpu.sync_copy(dbi_h.at[wid], dbi_v)
    ebase = wid * EPW

    for b in range(2):
        pltpu.async_copy(e0_h.at[dbi_v.at[b]], e0_v.at[b], s0)
        pltpu.async_copy(e1_h.at[dbi_v.at[b]], e1_v.at[b], s1)

    def chunk(i, carry):
        for b in range(2):
            j = i * 2 + b
            pltpu.async_copy(bs_h.at[dbi_v.at[j]], cbs_v.at[j], s2)
            pltpu.make_async_copy(e0_h.at[dbi_v.at[j]], e0_v.at[b],
                                  s0).wait()
            pltpu.make_async_copy(e1_h.at[dbi_v.at[j]], e1_v.at[b],
                                  s1).wait()
            for k in range(CG // 16):
                sl = pl.ds(k * 16, 16)
                ea = e0_v[b, sl]
                eb = e1_v[b, sl]
                ent = (ebase + j * CG + k * 16
                       + lax.broadcasted_iota(jnp.int32, (16,), 0))
                dev = jnp.right_shift(ent, 5)
                nid_v[j, sl] = jnp.where(ea != dev, ea, eb)
            nxt = j + 2

            @pl.when(nxt < CH)
            def _(b=b, nxt=nxt):
                pltpu.async_copy(e0_h.at[dbi_v.at[nxt]], e0_v.at[b], s0)
                pltpu.async_copy(e1_h.at[dbi_v.at[nxt]], e1_v.at[b], s1)

            @pl.when(j >= 32)
            def _(j=j):
                pltpu.make_async_copy(bs_h.at[dbi_v.at[j - 32]],
                                      cbs_v.at[j - 32], s2).wait()
        return carry

    lax.fori_loop(0, CH // 2, chunk, 0)
    pltpu.sync_copy(nid_v, nid_h.at[wid])

    def drain(j, carry):
        pltpu.make_async_copy(bs_h.at[dbi_v.at[j]], cbs_v.at[j], s2).wait()
        return carry

    lax.fori_loop(CH - 32, CH, drain, 0)
    pltpu.sync_copy(cbs_v, cbs_h.at[wid])


CH_TOT = NP // G
C_A = 80
C_B = (CH_TOT // 16) - C_A
CMAX = max(C_A, C_B)
FLUSH = 40


def _sc_gather_body(v_h, nid_h, out_h, idx_v, rows_v, ne_v, s0, s1):
    cid = lax.axis_index("c")
    sid = lax.axis_index("s")
    is_a = cid == 0
    cnt = jnp.where(is_a, C_A, C_B)
    start = jnp.where(is_a, sid * C_A, 16 * C_A + sid * C_B)
    sems = (s0, s1)

    @pl.when(cnt > 0)
    def _():
        @pl.when(is_a)
        def _():
            pltpu.sync_copy(nid_h.at[pl.ds(start, C_A)],
                            idx_v.at[pl.ds(0, C_A)])

        if C_B > 0:
            @pl.when(jnp.logical_not(is_a))
            def _():
                pltpu.sync_copy(nid_h.at[pl.ds(start, C_B)],
                                idx_v.at[pl.ds(0, C_B)])

        pltpu.async_copy(v_h.at[idx_v.at[0]], rows_v.at[0], s0)
        pltpu.async_copy(v_h.at[idx_v.at[1]], rows_v.at[1], s1)

        def outer(i, carry):
            for b in range(2):
                c = i * 2 + b
                pltpu.make_async_copy(v_h.at[idx_v.at[c]], rows_v.at[b],
                                      sems[b]).wait()
                fc = lax.rem(c, FLUSH)

                def dev(g, inner_carry, b=b, fc=fc):
                    r0 = g * DEG
                    accs = [rows_v[b, r0, pl.ds(k * 16, 16)]
                            for k in range(VR)]
                    for d in range(1, DEG):
                        for k in range(VR):
                            accs[k] = accs[k] + rows_v[b, r0 + d,
                                                       pl.ds(k * 16, 16)]
                    row = fc * G + g
                    for k in range(VR):
                        ne_v[row, pl.ds(k * 16, 16)] = accs[k]
                    return inner_carry

                lax.fori_loop(0, G, dev, 0)
                nxt = c + 2

                @pl.when(nxt < cnt)
                def _(b=b, nxt=nxt):
                    pltpu.async_copy(v_h.at[idx_v.at[nxt]], rows_v.at[b],
                                     sems[b])

                @pl.when(fc == FLUSH - 1)
                def _(c=c):
                    off = pl.multiple_of(
                        (start + c - (FLUSH - 1)) * G, 8)
                    pltpu.sync_copy(ne_v, out_h.at[pl.ds(off, FLUSH * G)])
            return carry

        lax.fori_loop(0, cnt // 2, outer, 0)


@functools.cache
def _sc_kernels():
    mesh = plsc.VectorSubcoreMesh(core_axis_name="c", subcore_axis_name="s",
                                  num_cores=2, num_subcores=16)
    prep = pl.kernel(
        _sc_prep_body,
        out_type=[jax.ShapeDtypeStruct((NW, CH, CG), jnp.int32),
                  jax.ShapeDtypeStruct((NW, CH, CG), jnp.float32)],
        mesh=mesh,
        scratch_types=[pltpu.VMEM((CH, CG), jnp.int32),
                       pltpu.VMEM((2, CG), jnp.int32),
                       pltpu.VMEM((2, CG), jnp.int32),
                       pltpu.VMEM((CH, CG), jnp.int32),
                       pltpu.VMEM((CH, CG), jnp.float32),
                       pltpu.SemaphoreType.DMA,
                       pltpu.SemaphoreType.DMA,
                       pltpu.SemaphoreType.DMA])
    gather = pl.kernel(
        _sc_gather_body,
        out_type=jax.ShapeDtypeStruct((NP, EMB), jnp.float32),
        name="seg_gather",
        mesh=mesh,
        scratch_types=[pltpu.VMEM((CMAX, CG), jnp.int32),
                       pltpu.VMEM((2, CG, EMB), jnp.float32),
                       pltpu.VMEM((FLUSH * G, EMB), jnp.float32),
                       pltpu.SemaphoreType.DMA,
                       pltpu.SemaphoreType.DMA])
    return prep, gather


def _tc_pre_body(convs, cbs_ref, ps_ref, w4t, b4t, w3, b3t, w1t, b1t, w2t,
                 b2t, w0, b0t, b5t, wb, ubt, f_ref, v1_ref):
    f32 = jnp.float32
    cbs = cbs_ref[...]
    sum_cbs = jnp.sum(cbs, axis=1, keepdims=True)
    be_in = jnp.zeros((cbs.shape[0], EMB), f32)
    for d in range(DEG):
        be_in = be_in + jnp.tanh(cbs[:, d:d + 1] * w4t[...] + b4t[...])
    be = jnp.tanh(
        lax.dot_general(be_in, w3[...], (((1,), (1,)), ((), ())),
                        preferred_element_type=f32) + b3t[...])
    ps = ps_ref[...]
    pe_in = jnp.zeros((cbs.shape[0], EMB), f32)
    for k in range(3):
        pe_in = pe_in + jnp.tanh(ps[:, k:k + 1] * w1t[...] + b1t[...])
    tb = jnp.tanh(sum_cbs * w2t[...] + b2t[...])
    pe = jnp.tanh(
        lax.dot_general(pe_in + 3.0 * tb, w0[...], (((1,), (1,)), ((), ())),
                        preferred_element_type=f32) + b0t[...])
    c0 = convs[0]
    c1 = convs[1]
    c2 = convs[2]
    cb = convs[3]
    f = c0 * pe + c1 * be + cb
    f_ref[...] = f
    emb = jnp.tanh(f + c2 * jnp.tanh(b5t[...]))
    g = lax.dot_general(emb, wb[...], (((1,), (1,)), ((), ())),
                        preferred_element_type=f32) + ubt[...]
    upd = g[:, EMB:2 * EMB]
    new = g[:, 2 * EMB:]
    v1_ref[...] = jnp.tanh(jax.nn.sigmoid(upd) * jnp.tanh(new))


def _tc_update_body(convs, v_ref, ne_ref, f_ref, w5, b5t, uw, ubt, vo_ref):
    f32 = jnp.float32
    ne = jnp.tanh(
        lax.dot_general(ne_ref[...], w5[...], (((1,), (1,)), ((), ())),
                        preferred_element_type=f32) + b5t[...])
    emb = jnp.tanh(f_ref[...] + convs[2] * ne)
    v = v_ref[...]
    x = jnp.concatenate([v, emb], axis=1)
    g = lax.dot_general(x, uw[...], (((1,), (1,)), ((), ())),
                        preferred_element_type=f32) + ubt[...]
    keep = g[:, :EMB]
    upd = g[:, EMB:2 * EMB]
    new = g[:, 2 * EMB:]
    vo_ref[...] = jnp.tanh(v * jax.nn.sigmoid(keep)
                           + jax.nn.sigmoid(upd) * jnp.tanh(new))


def _tc_grid_body(v_ref, gw, gbt, out_ref):
    rid = lax.broadcasted_iota(jnp.int32, (NP, 1), 0)
    vm = jnp.where(rid < N, v_ref[...], 0.0)
    parts = [jnp.sum(vm[i * 320:(i + 1) * 320], axis=0, keepdims=True)
             for i in range(32)]
    while len(parts) > 1:
        parts = [parts[i] + parts[i + 1] for i in range(0, len(parts), 2)]
    s = parts[0]
    out_ref[...] = lax.dot_general(s, gw[...], (((1,), (1,)), ((), ())),
                                   preferred_element_type=jnp.float32) + gbt[...]


def _row_spec(cols):
    return pl.BlockSpec((NB, cols), lambda i: (i, 0))


def _full_spec(shape):
    return pl.BlockSpec(shape, lambda i: tuple(0 for _ in shape))


_SMEM_SPEC = pl.BlockSpec(memory_space=pltpu.SMEM)

_GRID = NP // NB

_tc_pre = pl.pallas_call(
    _tc_pre_body,
    grid=(_GRID,),
    in_specs=[_SMEM_SPEC, _row_spec(DEG), _row_spec(3),
              _full_spec((1, EMB)), _full_spec((1, EMB)),
              _full_spec((EMB, EMB)), _full_spec((1, EMB)),
              _full_spec((1, EMB)), _full_spec((1, EMB)),
              _full_spec((1, EMB)), _full_spec((1, EMB)),
              _full_spec((EMB, EMB)), _full_spec((1, EMB)),
              _full_spec((1, EMB)), _full_spec((3 * EMB, EMB)),
              _full_spec((1, 3 * EMB))],
    out_specs=[_row_spec(EMB), _row_spec(EMB)],
    out_shape=[jax.ShapeDtypeStruct((NP, EMB), jnp.float32),
               jax.ShapeDtypeStruct((NP, EMB), jnp.float32)],
)

_tc_update = pl.pallas_call(
    _tc_update_body,
    grid=(_GRID,),
    in_specs=[_SMEM_SPEC, _row_spec(EMB), _row_spec(EMB), _row_spec(EMB),
              _full_spec((EMB, EMB)), _full_spec((1, EMB)),
              _full_spec((3 * EMB, 2 * EMB)), _full_spec((1, 3 * EMB))],
    out_specs=_row_spec(EMB),
    out_shape=jax.ShapeDtypeStruct((NP, EMB), jnp.float32),
)

_tc_grid = pl.pallas_call(
    _tc_grid_body,
    out_shape=jax.ShapeDtypeStruct((1, EMB), jnp.float32),
)


def kernel(protector_state, breaker_state, device_breaker_ids, breakers,
           W0, b0, W1, b1, W2, b2, W3, b3, W4, b4, W5, b5,
           conv_w, conv_b, update_W, update_b, grid_W, grid_b):
    f32 = jnp.float32
    dbi = device_breaker_ids.astype(jnp.int32)
    pad_ids = (jnp.arange((NP - N) * DEG, dtype=jnp.int32) * 97
               % jnp.int32(E_TOT)).reshape(NP - N, DEG)
    dbi_pad = jnp.concatenate([dbi, pad_ids], axis=0)
    dbi_r = dbi_pad.reshape(NW, CH, CG)
    e0 = breakers[:, 0].astype(jnp.int32)
    e1 = breakers[:, 1].astype(jnp.int32)
    ps_pad = jnp.pad(protector_state.astype(f32), ((0, NP - N), (0, 0)))

    _sc_prep, _sc_gather = _sc_kernels()
    nid_r, cbs_flat = _sc_prep(dbi_r, e0, e1, breaker_state.astype(f32))
    nid_r = nid_r.reshape(CH_TOT, CG)
    cbs = cbs_flat.reshape(NP, DEG)

    w1t = W1.T.astype(f32)
    w2t = W2.T.astype(f32)
    w4t = W4.T.astype(f32)
    b0t = b0.reshape(1, EMB)
    b1t = b1.reshape(1, EMB)
    b2t = b2.reshape(1, EMB)
    b3t = b3.reshape(1, EMB)
    b4t = b4.reshape(1, EMB)
    b5t = b5.reshape(1, EMB)
    ubt = update_b.reshape(1, 3 * EMB)
    gbt = grid_b.reshape(1, EMB)
    wb = update_W[:, EMB:]
    convs = jnp.concatenate([conv_w.reshape(3), conv_b.reshape(1)]).astype(f32)

    F, V = _tc_pre(convs, cbs, ps_pad, w4t, b4t, W3, b3t, w1t, b1t, w2t,
                   b2t, W0, b0t, b5t, wb, ubt)
    for _ in range(2):
        ne_raw = _sc_gather(V, nid_r)
        V = _tc_update(convs, V, ne_raw, F, W5, b5t, update_W, ubt)
    grid_emb = _tc_grid(V, grid_W, gbt)
    return V[:N], grid_emb.reshape(EMB)


# speedup vs baseline: 3.2359x; 1.0246x over previous
import functools

import jax
import jax.numpy as jnp
from jax import lax
from jax.experimental import pallas as pl
from jax.experimental.pallas import tpu as pltpu
from jax.experimental.pallas import tpu_sc as plsc

N = 10000
DEG = 32
EMB = 128
E_TOT = 160000

NW = 32
NP = 10240
DPW = NP // NW
EPW = DPW * DEG
CG = 128
G = CG // DEG
CH = DPW // G
NB = 512
VR = EMB // 16

def _wid():
    return lax.axis_index("s") * 2 + lax.axis_index("c")


def _sc_prep_body(dbi_h, e0_h, e1_h, bs_h, nid_h, cbs_h,
                  dbi_v, e0_v, e1_v, nid_v, cbs_v, s0, s1, s2):
    wid = _wid()
    pltpu.sync_copy(dbi_h.at[wid], dbi_v)
    ebase = wid * EPW

    for b in range(2):
        pltpu.async_copy(e0_h.at[dbi_v.at[b]], e0_v.at[b], s0)
        pltpu.async_copy(e1_h.at[dbi_v.at[b]], e1_v.at[b], s1)

    def chunk(i, carry):
        for b in range(2):
            j = i * 2 + b
            pltpu.async_copy(bs_h.at[dbi_v.at[j]], cbs_v.at[j], s2)
            pltpu.make_async_copy(e0_h.at[dbi_v.at[j]], e0_v.at[b],
                                  s0).wait()
            pltpu.make_async_copy(e1_h.at[dbi_v.at[j]], e1_v.at[b],
                                  s1).wait()
            for k in range(CG // 16):
                sl = pl.ds(k * 16, 16)
                ea = e0_v[b, sl]
                eb = e1_v[b, sl]
                ent = (ebase + j * CG + k * 16
                       + lax.broadcasted_iota(jnp.int32, (16,), 0))
                dev = jnp.right_shift(ent, 5)
                nid_v[j, sl] = jnp.where(ea != dev, ea, eb)
            nxt = j + 2

            @pl.when(nxt < CH)
            def _(b=b, nxt=nxt):
                pltpu.async_copy(e0_h.at[dbi_v.at[nxt]], e0_v.at[b], s0)
                pltpu.async_copy(e1_h.at[dbi_v.at[nxt]], e1_v.at[b], s1)

            @pl.when(j >= 32)
            def _(j=j):
                pltpu.make_async_copy(bs_h.at[dbi_v.at[j - 32]],
                                      cbs_v.at[j - 32], s2).wait()
        return carry

    lax.fori_loop(0, CH // 2, chunk, 0)
    pltpu.sync_copy(nid_v, nid_h.at[wid])

    def drain(j, carry):
        pltpu.make_async_copy(bs_h.at[dbi_v.at[j]], cbs_v.at[j], s2).wait()
        return carry

    lax.fori_loop(CH - 32, CH, drain, 0)
    pltpu.sync_copy(cbs_v, cbs_h.at[wid])


CH_TOT = NP // G
C_A = 80
C_B = (CH_TOT // 16) - C_A
CMAX = max(C_A, C_B)
FLUSH = 40


NBUF = 4


def _sc_gather_body(v_h, nid_h, out_h, idx_v, rows_v, ne_v, s0, s1, s2, s3):
    cid = lax.axis_index("c")
    sid = lax.axis_index("s")
    is_a = cid == 0
    cnt = jnp.where(is_a, C_A, C_B)
    start = jnp.where(is_a, sid * C_A, 16 * C_A + sid * C_B)
    sems = (s0, s1, s2, s3)

    @pl.when(cnt > 0)
    def _():
        @pl.when(is_a)
        def _():
            pltpu.sync_copy(nid_h.at[pl.ds(start, C_A)],
                            idx_v.at[pl.ds(0, C_A)])

        if C_B > 0:
            @pl.when(jnp.logical_not(is_a))
            def _():
                pltpu.sync_copy(nid_h.at[pl.ds(start, C_B)],
                                idx_v.at[pl.ds(0, C_B)])

        for b in range(NBUF):
            pltpu.async_copy(v_h.at[idx_v.at[b]], rows_v.at[b], sems[b])

        def outer(i, carry):
            for b in range(NBUF):
                c = i * NBUF + b
                pltpu.make_async_copy(v_h.at[idx_v.at[c]], rows_v.at[b],
                                      sems[b]).wait()
                fc = lax.rem(c, FLUSH)

                def dev(g, inner_carry, b=b, fc=fc):
                    r0 = g * DEG
                    accs = [rows_v[b, r0, pl.ds(k * 16, 16)]
                            for k in range(VR)]
                    for d in range(1, DEG):
                        for k in range(VR):
                            accs[k] = accs[k] + rows_v[b, r0 + d,
                                                       pl.ds(k * 16, 16)]
                    row = fc * G + g
                    for k in range(VR):
                        ne_v[row, pl.ds(k * 16, 16)] = accs[k]
                    return inner_carry

                lax.fori_loop(0, G, dev, 0)
                nxt = c + NBUF

                @pl.when(nxt < cnt)
                def _(b=b, nxt=nxt):
                    pltpu.async_copy(v_h.at[idx_v.at[nxt]], rows_v.at[b],
                                     sems[b])

                @pl.when(fc == FLUSH - 1)
                def _(c=c):
                    off = pl.multiple_of(
                        (start + c - (FLUSH - 1)) * G, 8)
                    pltpu.sync_copy(ne_v, out_h.at[pl.ds(off, FLUSH * G)])
            return carry

        lax.fori_loop(0, cnt // NBUF, outer, 0)


@functools.cache
def _sc_kernels():
    mesh = plsc.VectorSubcoreMesh(core_axis_name="c", subcore_axis_name="s",
                                  num_cores=2, num_subcores=16)
    prep = pl.kernel(
        _sc_prep_body,
        out_type=[jax.ShapeDtypeStruct((NW, CH, CG), jnp.int32),
                  jax.ShapeDtypeStruct((NW, CH, CG), jnp.float32)],
        mesh=mesh,
        scratch_types=[pltpu.VMEM((CH, CG), jnp.int32),
                       pltpu.VMEM((2, CG), jnp.int32),
                       pltpu.VMEM((2, CG), jnp.int32),
                       pltpu.VMEM((CH, CG), jnp.int32),
                       pltpu.VMEM((CH, CG), jnp.float32),
                       pltpu.SemaphoreType.DMA,
                       pltpu.SemaphoreType.DMA,
                       pltpu.SemaphoreType.DMA])
    gather = pl.kernel(
        _sc_gather_body,
        out_type=jax.ShapeDtypeStruct((NP, EMB), jnp.float32),
        name="seg_gather",
        mesh=mesh,
        scratch_types=[pltpu.VMEM((CMAX, CG), jnp.int32),
                       pltpu.VMEM((NBUF, CG, EMB), jnp.float32),
                       pltpu.VMEM((FLUSH * G, EMB), jnp.float32),
                       pltpu.SemaphoreType.DMA,
                       pltpu.SemaphoreType.DMA,
                       pltpu.SemaphoreType.DMA,
                       pltpu.SemaphoreType.DMA])
    return prep, gather


def _tc_pre_body(convs, cbs_ref, ps_ref, w4t, b4t, w3, b3t, w1t, b1t, w2t,
                 b2t, w0, b0t, b5t, wb, ubt, f_ref, v1_ref):
    f32 = jnp.float32
    cbs = cbs_ref[...]
    sum_cbs = jnp.sum(cbs, axis=1, keepdims=True)
    be_in = jnp.zeros((cbs.shape[0], EMB), f32)
    for d in range(DEG):
        be_in = be_in + jnp.tanh(cbs[:, d:d + 1] * w4t[...] + b4t[...])
    be = jnp.tanh(
        lax.dot_general(be_in, w3[...], (((1,), (1,)), ((), ())),
                        preferred_element_type=f32) + b3t[...])
    ps = ps_ref[...]
    pe_in = jnp.zeros((cbs.shape[0], EMB), f32)
    for k in range(3):
        pe_in = pe_in + jnp.tanh(ps[:, k:k + 1] * w1t[...] + b1t[...])
    tb = jnp.tanh(sum_cbs * w2t[...] + b2t[...])
    pe = jnp.tanh(
        lax.dot_general(pe_in + 3.0 * tb, w0[...], (((1,), (1,)), ((), ())),
                        preferred_element_type=f32) + b0t[...])
    c0 = convs[0]
    c1 = convs[1]
    c2 = convs[2]
    cb = convs[3]
    f = c0 * pe + c1 * be + cb
    f_ref[...] = f
    emb = jnp.tanh(f + c2 * jnp.tanh(b5t[...]))
    g = lax.dot_general(emb, wb[...], (((1,), (1,)), ((), ())),
                        preferred_element_type=f32) + ubt[...]
    upd = g[:, EMB:2 * EMB]
    new = g[:, 2 * EMB:]
    v1_ref[...] = jnp.tanh(jax.nn.sigmoid(upd) * jnp.tanh(new))


def _tc_update_body(convs, v_ref, ne_ref, f_ref, w5, b5t, uw, ubt, vo_ref):
    f32 = jnp.float32
    ne = jnp.tanh(
        lax.dot_general(ne_ref[...], w5[...], (((1,), (1,)), ((), ())),
                        preferred_element_type=f32) + b5t[...])
    emb = jnp.tanh(f_ref[...] + convs[2] * ne)
    v = v_ref[...]
    x = jnp.concatenate([v, emb], axis=1)
    g = lax.dot_general(x, uw[...], (((1,), (1,)), ((), ())),
                        preferred_element_type=f32) + ubt[...]
    keep = g[:, :EMB]
    upd = g[:, EMB:2 * EMB]
    new = g[:, 2 * EMB:]
    vo_ref[...] = jnp.tanh(v * jax.nn.sigmoid(keep)
                           + jax.nn.sigmoid(upd) * jnp.tanh(new))


def _tc_grid_body(v_ref, gw, gbt, out_ref):
    rid = lax.broadcasted_iota(jnp.int32, (NP, 1), 0)
    vm = jnp.where(rid < N, v_ref[...], 0.0)
    parts = [jnp.sum(vm[i * 320:(i + 1) * 320], axis=0, keepdims=True)
             for i in range(32)]
    while len(parts) > 1:
        parts = [parts[i] + parts[i + 1] for i in range(0, len(parts), 2)]
    s = parts[0]
    out_ref[...] = lax.dot_general(s, gw[...], (((1,), (1,)), ((), ())),
                                   preferred_element_type=jnp.float32) + gbt[...]


def _row_spec(cols):
    return pl.BlockSpec((NB, cols), lambda i: (i, 0))


def _full_spec(shape):
    return pl.BlockSpec(shape, lambda i: tuple(0 for _ in shape))


_SMEM_SPEC = pl.BlockSpec(memory_space=pltpu.SMEM)

_GRID = NP // NB

_tc_pre = pl.pallas_call(
    _tc_pre_body,
    grid=(_GRID,),
    in_specs=[_SMEM_SPEC, _row_spec(DEG), _row_spec(3),
              _full_spec((1, EMB)), _full_spec((1, EMB)),
              _full_spec((EMB, EMB)), _full_spec((1, EMB)),
              _full_spec((1, EMB)), _full_spec((1, EMB)),
              _full_spec((1, EMB)), _full_spec((1, EMB)),
              _full_spec((EMB, EMB)), _full_spec((1, EMB)),
              _full_spec((1, EMB)), _full_spec((3 * EMB, EMB)),
              _full_spec((1, 3 * EMB))],
    out_specs=[_row_spec(EMB), _row_spec(EMB)],
    out_shape=[jax.ShapeDtypeStruct((NP, EMB), jnp.float32),
               jax.ShapeDtypeStruct((NP, EMB), jnp.float32)],
)

_tc_update = pl.pallas_call(
    _tc_update_body,
    grid=(_GRID,),
    in_specs=[_SMEM_SPEC, _row_spec(EMB), _row_spec(EMB), _row_spec(EMB),
              _full_spec((EMB, EMB)), _full_spec((1, EMB)),
              _full_spec((3 * EMB, 2 * EMB)), _full_spec((1, 3 * EMB))],
    out_specs=_row_spec(EMB),
    out_shape=jax.ShapeDtypeStruct((NP, EMB), jnp.float32),
)

_tc_grid = pl.pallas_call(
    _tc_grid_body,
    out_shape=jax.ShapeDtypeStruct((1, EMB), jnp.float32),
)


def kernel(protector_state, breaker_state, device_breaker_ids, breakers,
           W0, b0, W1, b1, W2, b2, W3, b3, W4, b4, W5, b5,
           conv_w, conv_b, update_W, update_b, grid_W, grid_b):
    f32 = jnp.float32
    dbi = device_breaker_ids.astype(jnp.int32)
    pad_ids = (jnp.arange((NP - N) * DEG, dtype=jnp.int32) * 97
               % jnp.int32(E_TOT)).reshape(NP - N, DEG)
    dbi_pad = jnp.concatenate([dbi, pad_ids], axis=0)
    dbi_r = dbi_pad.reshape(NW, CH, CG)
    e0 = breakers[:, 0].astype(jnp.int32)
    e1 = breakers[:, 1].astype(jnp.int32)
    ps_pad = jnp.pad(protector_state.astype(f32), ((0, NP - N), (0, 0)))

    _sc_prep, _sc_gather = _sc_kernels()
    nid_r, cbs_flat = _sc_prep(dbi_r, e0, e1, breaker_state.astype(f32))
    nid_r = nid_r.reshape(CH_TOT, CG)
    cbs = cbs_flat.reshape(NP, DEG)

    w1t = W1.T.astype(f32)
    w2t = W2.T.astype(f32)
    w4t = W4.T.astype(f32)
    b0t = b0.reshape(1, EMB)
    b1t = b1.reshape(1, EMB)
    b2t = b2.reshape(1, EMB)
    b3t = b3.reshape(1, EMB)
    b4t = b4.reshape(1, EMB)
    b5t = b5.reshape(1, EMB)
    ubt = update_b.reshape(1, 3 * EMB)
    gbt = grid_b.reshape(1, EMB)
    wb = update_W[:, EMB:]
    convs = jnp.concatenate([conv_w.reshape(3), conv_b.reshape(1)]).astype(f32)

    F, V = _tc_pre(convs, cbs, ps_pad, w4t, b4t, W3, b3t, w1t, b1t, w2t,
                   b2t, W0, b0t, b5t, wb, ubt)
    for _ in range(2):
        ne_raw = _sc_gather(V, nid_r)
        V = _tc_update(convs, V, ne_raw, F, W5, b5t, update_W, ubt)
    grid_emb = _tc_grid(V, grid_W, gbt)
    return V[:N], grid_emb.reshape(EMB)
